# baseline (speedup 1.0000x reference)
import functools

import jax
import jax.numpy as jnp
from jax import lax
from jax.experimental import pallas as pl
from jax.experimental.pallas import tpu as pltpu
from jax.experimental.pallas import tpu_sc as plsc

N = 10000
E = 320000
D = 128
DE = 16
DU = 16
G = 128
OUT = 128
H = OUT // 2

NC = 2
NS = 16
EPT = E // NS
CHUNK = 80
NCHUNK = EPT // CHUNK
RD = 40
NRD = N // RD

_HIGH = jax.lax.Precision.HIGHEST


def _precompute_body(x_ref, u_ref, batch_ref, w1_ref, b1_ref, w2_ref, b2_ref,
                     p_ref, q_ref, r_ref):
    x = x_ref[...]
    bat = batch_ref[...]
    gidx = lax.broadcasted_iota(jnp.int32, (x.shape[0], G), 1)
    onehot = (bat == gidx).astype(jnp.float32)
    uu = jnp.dot(onehot, u_ref[...], preferred_element_type=jnp.float32,
                 precision=_HIGH)
    w1 = w1_ref[...]
    w2 = w2_ref[...]
    p = (jnp.dot(x, w1[:D], preferred_element_type=jnp.float32, precision=_HIGH)
         + jnp.dot(uu, w1[2 * D + DE:], preferred_element_type=jnp.float32,
                   precision=_HIGH)
         + b1_ref[...])
    q = jnp.dot(x, w1[D:2 * D], preferred_element_type=jnp.float32,
                precision=_HIGH)
    r = (jnp.dot(x, w2[:D], preferred_element_type=jnp.float32, precision=_HIGH)
         + jnp.dot(uu, w2[D + OUT:], preferred_element_type=jnp.float32,
                   precision=_HIGH)
         + b2_ref[...])
    p_ref[0] = p[:, :H]
    p_ref[1] = p[:, H:]
    q_ref[0] = q[:, :H]
    q_ref[1] = q[:, H:]
    r_ref[...] = r


def _ew_body(eat8_ref, w1e_ref, ew_ref):
    w1e = w1e_ref[...]
    blkm = ew_ref.shape[1] // 4
    dn = (((0,), (0,)), ((), ()))
    for core in range(NC):
        wc = w1e[:, core * H:(core + 1) * H]
        cols = [lax.dot_general(eat8_ref[:, 0, j, :], wc, dn,
                                preferred_element_type=jnp.float32)
                for j in range(8)]
        packed = jnp.concatenate(cols, axis=1)
        ew_ref[core] = packed.reshape(blkm * 4, 128)


def _final_body(g_ref, r_ref, w2b_ref, out_ref):
    grouped = jnp.concatenate([g_ref[0], g_ref[1]], axis=1)
    acc = jnp.dot(grouped, w2b_ref[...], preferred_element_type=jnp.float32,
                  precision=_HIGH)
    out_ref[...] = jnp.maximum(acc + r_ref[...], 0.0)


def _sc_body(p_hbm, q_hbm, ew_hbm, dst_hbm, src_hbm, g_hbm,
             idxd, idxs, buf_e, buf_p, buf_q, buf_o, zbuf, acc,
             sem_e, sem_p, sem_q, sem_s):
    c = lax.axis_index("c")
    s = lax.axis_index("s")

    def _zrow(r, carry):
        for cc in range(H // 16):
            zbuf[r, pl.ds(cc * 16, 16)] = jnp.zeros((16,), jnp.float32)
        return carry
    lax.fori_loop(0, RD, _zrow, 0)
    for t in range((NRD + NS - 1) // NS):
        cidx = t * NS + s
        @pl.when(cidx < NRD)
        def _():
            pltpu.sync_copy(zbuf, acc.at[pl.ds(cidx * RD, RD)])
    plsc.subcore_barrier()

    pltpu.sync_copy(dst_hbm.at[s], idxd)
    pltpu.sync_copy(src_hbm.at[s], idxs)

    pc = p_hbm.at[c]
    qc = q_hbm.at[c]
    ewc = ew_hbm.at[c]

    def _issue_in(j, b):
        rbase = (s * EPT + j * CHUNK) // 2
        pltpu.async_copy(ewc.at[pl.ds(rbase, CHUNK // 2)], buf_e[b], sem_e[b])
        pltpu.async_copy(pc.at[idxd.at[j]], buf_p[b], sem_p[b])
        pltpu.async_copy(qc.at[idxs.at[j]], buf_q[b], sem_q[b])

    def _wait_in(b):
        pltpu.make_async_copy(ewc.at[pl.ds(0, CHUNK // 2)], buf_e[b], sem_e[b]).wait()
        pltpu.make_async_copy(pc.at[pl.ds(0, CHUNK)], buf_p[b], sem_p[b]).wait()
        pltpu.make_async_copy(qc.at[pl.ds(0, CHUNK)], buf_q[b], sem_q[b]).wait()

    for b in range(2):
        _issue_in(b, b)

    def _outer(j0, carry):
        for b in range(2):
            j = j0 * 2 + b
            _wait_in(b)

            @pl.when(j0 > 0)
            def _():
                pltpu.make_async_copy(pc.at[pl.ds(0, CHUNK)], buf_o[b],
                                      sem_s[b]).wait()

            def _row(rr, carry2):
                for half in range(2):
                    e = 2 * rr + half
                    for cc in range(H // 16):
                        sl = pl.ds(cc * 16, 16)
                        v = (buf_e[b][rr, pl.ds(half * H + cc * 16, 16)]
                             + buf_p[b][e, sl] + buf_q[b][e, sl])
                        buf_o[b][e, sl] = jnp.maximum(v, 0.0)
                return carry2
            lax.fori_loop(0, CHUNK // 2, _row, 0)

            pltpu.async_copy(buf_o[b], acc.at[idxd.at[j]], sem_s[b], add=True)

            @pl.when(j + 2 < NCHUNK)
            def _():
                _issue_in(j + 2, b)
        return carry
    lax.fori_loop(0, NCHUNK // 2, _outer, 0)
    for b in range(2):
        pltpu.make_async_copy(pc.at[pl.ds(0, CHUNK)], buf_o[b], sem_s[b]).wait()
    plsc.subcore_barrier()

    for t in range((NRD + NS - 1) // NS):
        cidx = t * NS + s
        @pl.when(cidx < NRD)
        def _():
            pltpu.sync_copy(acc.at[pl.ds(cidx * RD, RD)], zbuf)
            pltpu.sync_copy(zbuf, g_hbm.at[c, pl.ds(cidx * RD, RD)])


@jax.jit
def kernel(x, edge_index, edge_attr, u, batch, W1, b1, W2, b2):
    E8 = E // 8
    src = (edge_index[0].reshape(8, E8).transpose(1, 0)
           .reshape(NS, NCHUNK, CHUNK))
    dst = (edge_index[1].reshape(8, E8).transpose(1, 0)
           .reshape(NS, NCHUNK, CHUNK))

    NBLK = 1000
    p, q, r = pl.pallas_call(
        _precompute_body,
        grid=(N // NBLK,),
        in_specs=[
            pl.BlockSpec((NBLK, D), lambda i: (i, 0)),
            pl.BlockSpec((G, DU), lambda i: (0, 0)),
            pl.BlockSpec((NBLK, 1), lambda i: (i, 0)),
            pl.BlockSpec((2 * D + DE + DU, OUT), lambda i: (0, 0)),
            pl.BlockSpec((1, OUT), lambda i: (0, 0)),
            pl.BlockSpec((D + OUT + DU, OUT), lambda i: (0, 0)),
            pl.BlockSpec((1, OUT), lambda i: (0, 0)),
        ],
        out_specs=[
            pl.BlockSpec((NC, NBLK, H), lambda i: (0, i, 0)),
            pl.BlockSpec((NC, NBLK, H), lambda i: (0, i, 0)),
            pl.BlockSpec((NBLK, OUT), lambda i: (i, 0)),
        ],
        out_shape=[
            jax.ShapeDtypeStruct((NC, N, H), jnp.float32),
            jax.ShapeDtypeStruct((NC, N, H), jnp.float32),
            jax.ShapeDtypeStruct((N, OUT), jnp.float32),
        ],
    )(x, u, batch.reshape(N, 1), W1, b1.reshape(1, OUT), W2, b2.reshape(1, OUT))

    EBLK8 = 2000
    NI = E8 // EBLK8
    eat8 = (edge_attr.T.reshape(DE, 8, NI, EBLK8).transpose(0, 2, 1, 3))
    w1e = W1[2 * D:2 * D + DE]

    ew2 = pl.pallas_call(
        _ew_body,
        grid=(NI,),
        in_specs=[
            pl.BlockSpec((DE, 1, 8, EBLK8), lambda i: (0, i, 0, 0)),
            pl.BlockSpec((DE, OUT), lambda i: (0, 0)),
        ],
        out_specs=pl.BlockSpec((NC, EBLK8 * 4, 2 * H), lambda i: (0, i, 0)),
        out_shape=jax.ShapeDtypeStruct((NC, E // 2, 2 * H), jnp.float32),
    )(eat8, w1e)

    mesh = plsc.VectorSubcoreMesh(core_axis_name="c", subcore_axis_name="s")
    g = pl.kernel(
        _sc_body,
        out_type=jax.ShapeDtypeStruct((NC, N, H), jnp.float32),
        mesh=mesh,
        compiler_params=pltpu.CompilerParams(use_tc_tiling_on_sc=False),
        scratch_types=[
            pltpu.VMEM((NCHUNK, CHUNK), jnp.int32),
            pltpu.VMEM((NCHUNK, CHUNK), jnp.int32),
            [pltpu.VMEM((CHUNK // 2, 2 * H), jnp.float32)] * 2,
            [pltpu.VMEM((CHUNK, H), jnp.float32)] * 2,
            [pltpu.VMEM((CHUNK, H), jnp.float32)] * 2,
            [pltpu.VMEM((CHUNK, H), jnp.float32)] * 2,
            pltpu.VMEM((RD, H), jnp.float32),
            pltpu.VMEM_SHARED((N, H), jnp.float32),
            [pltpu.SemaphoreType.DMA] * 2,
            [pltpu.SemaphoreType.DMA] * 2,
            [pltpu.SemaphoreType.DMA] * 2,
            [pltpu.SemaphoreType.DMA] * 2,
        ],
    )(p, q, ew2, dst, src)

    out = pl.pallas_call(
        _final_body,
        out_shape=jax.ShapeDtypeStruct((N, OUT), jnp.float32),
    )(g, r, W2[D:D + OUT])
    return out


# speedup vs baseline: 1.4085x; 1.4085x over previous
import functools

import jax
import jax.numpy as jnp
from jax import lax
from jax.experimental import pallas as pl
from jax.experimental.pallas import tpu as pltpu
from jax.experimental.pallas import tpu_sc as plsc

N = 10000
E = 320000
D = 128
DE = 16
DU = 16
G = 128
OUT = 128
H = OUT // 2

NC = 2
NS = 16
EPT = E // NS
CHUNK = 80
NCHUNK = EPT // CHUNK
RD = 40
NRD = N // RD

_HIGH = jax.lax.Precision.HIGHEST


def _precompute_body(x_ref, u_ref, batch_ref, w1_ref, b1_ref, w2_ref, b2_ref,
                     p_ref, q_ref, r_ref):
    x = x_ref[...]
    bat = batch_ref[...]
    gidx = lax.broadcasted_iota(jnp.int32, (x.shape[0], G), 1)
    onehot = (bat == gidx).astype(jnp.float32)
    uu = jnp.dot(onehot, u_ref[...], preferred_element_type=jnp.float32,
                 precision=_HIGH)
    w1 = w1_ref[...]
    w2 = w2_ref[...]
    p = (jnp.dot(x, w1[:D], preferred_element_type=jnp.float32, precision=_HIGH)
         + jnp.dot(uu, w1[2 * D + DE:], preferred_element_type=jnp.float32,
                   precision=_HIGH)
         + b1_ref[...])
    q = jnp.dot(x, w1[D:2 * D], preferred_element_type=jnp.float32,
                precision=_HIGH)
    r = (jnp.dot(x, w2[:D], preferred_element_type=jnp.float32, precision=_HIGH)
         + jnp.dot(uu, w2[D + OUT:], preferred_element_type=jnp.float32,
                   precision=_HIGH)
         + b2_ref[...])
    p_ref[0] = p[:, :H]
    p_ref[1] = p[:, H:]
    q_ref[0] = q[:, :H]
    q_ref[1] = q[:, H:]
    r_ref[...] = r


def _ew_body(eat8_ref, w1e_ref, ew_ref):
    dn = (((0,), (0,)), ((), ()))
    ew_ref[...] = lax.dot_general(eat8_ref[...], w1e_ref[...], dn,
                                  preferred_element_type=jnp.float32)


def _final_body(g_ref, r_ref, w2b_ref, out_ref):
    grouped = jnp.concatenate([g_ref[0], g_ref[1]], axis=1)
    acc = jnp.dot(grouped, w2b_ref[...], preferred_element_type=jnp.float32,
                  precision=_HIGH)
    out_ref[...] = jnp.maximum(acc + r_ref[...], 0.0)


def _sc_body(p_hbm, q_hbm, ew_hbm, dst_hbm, src_hbm, g_hbm,
             idxd, idxs, buf_e, buf_p, buf_q, buf_o, zbuf, acc,
             sem_e, sem_p, sem_q, sem_s):
    c = lax.axis_index("c")
    s = lax.axis_index("s")

    def _zrow(r, carry):
        for cc in range(H // 16):
            zbuf[r, pl.ds(cc * 16, 16)] = jnp.zeros((16,), jnp.float32)
        return carry
    lax.fori_loop(0, RD, _zrow, 0)
    for t in range((NRD + NS - 1) // NS):
        cidx = t * NS + s
        @pl.when(cidx < NRD)
        def _():
            pltpu.sync_copy(zbuf, acc.at[pl.ds(cidx * RD, RD)])
    plsc.subcore_barrier()

    pltpu.sync_copy(dst_hbm.at[s], idxd)
    pltpu.sync_copy(src_hbm.at[s], idxs)

    pc = p_hbm.at[c]
    qc = q_hbm.at[c]

    def _issue_in(j, b):
        ebase = s * EPT + j * CHUNK
        pltpu.async_copy(
            ew_hbm.at[pl.ds(ebase, CHUNK), pl.ds(c * H, H)], buf_e[b], sem_e[b])
        pltpu.async_copy(pc.at[idxd.at[j]], buf_p[b], sem_p[b])
        pltpu.async_copy(qc.at[idxs.at[j]], buf_q[b], sem_q[b])

    def _wait_in(b):
        pltpu.make_async_copy(pc.at[pl.ds(0, CHUNK)], buf_e[b], sem_e[b]).wait()
        pltpu.make_async_copy(pc.at[pl.ds(0, CHUNK)], buf_p[b], sem_p[b]).wait()
        pltpu.make_async_copy(pc.at[pl.ds(0, CHUNK)], buf_q[b], sem_q[b]).wait()

    for b in range(2):
        _issue_in(b, b)

    def _outer(j0, carry):
        for b in range(2):
            j = j0 * 2 + b
            _wait_in(b)

            @pl.when(j0 > 0)
            def _():
                pltpu.make_async_copy(pc.at[pl.ds(0, CHUNK)], buf_o[b],
                                      sem_s[b]).wait()

            def _row(e, carry2):
                for cc in range(H // 16):
                    sl = pl.ds(cc * 16, 16)
                    v = (buf_e[b][e, sl]
                         + buf_p[b][e, sl] + buf_q[b][e, sl])
                    buf_o[b][e, sl] = jnp.maximum(v, 0.0)
                return carry2
            lax.fori_loop(0, CHUNK, _row, 0)

            pltpu.async_copy(buf_o[b], acc.at[idxd.at[j]], sem_s[b], add=True)

            @pl.when(j + 2 < NCHUNK)
            def _():
                _issue_in(j + 2, b)
        return carry
    lax.fori_loop(0, NCHUNK // 2, _outer, 0)
    for b in range(2):
        pltpu.make_async_copy(pc.at[pl.ds(0, CHUNK)], buf_o[b], sem_s[b]).wait()
    plsc.subcore_barrier()

    for t in range((NRD + NS - 1) // NS):
        cidx = t * NS + s
        @pl.when(cidx < NRD)
        def _():
            pltpu.sync_copy(acc.at[pl.ds(cidx * RD, RD)], zbuf)
            pltpu.sync_copy(zbuf, g_hbm.at[c, pl.ds(cidx * RD, RD)])


@jax.jit
def kernel(x, edge_index, edge_attr, u, batch, W1, b1, W2, b2):
    src = edge_index[0].reshape(NS, NCHUNK, CHUNK)
    dst = edge_index[1].reshape(NS, NCHUNK, CHUNK)

    NBLK = 1000
    p, q, r = pl.pallas_call(
        _precompute_body,
        grid=(N // NBLK,),
        in_specs=[
            pl.BlockSpec((NBLK, D), lambda i: (i, 0)),
            pl.BlockSpec((G, DU), lambda i: (0, 0)),
            pl.BlockSpec((NBLK, 1), lambda i: (i, 0)),
            pl.BlockSpec((2 * D + DE + DU, OUT), lambda i: (0, 0)),
            pl.BlockSpec((1, OUT), lambda i: (0, 0)),
            pl.BlockSpec((D + OUT + DU, OUT), lambda i: (0, 0)),
            pl.BlockSpec((1, OUT), lambda i: (0, 0)),
        ],
        out_specs=[
            pl.BlockSpec((NC, NBLK, H), lambda i: (0, i, 0)),
            pl.BlockSpec((NC, NBLK, H), lambda i: (0, i, 0)),
            pl.BlockSpec((NBLK, OUT), lambda i: (i, 0)),
        ],
        out_shape=[
            jax.ShapeDtypeStruct((NC, N, H), jnp.float32),
            jax.ShapeDtypeStruct((NC, N, H), jnp.float32),
            jax.ShapeDtypeStruct((N, OUT), jnp.float32),
        ],
    )(x, u, batch.reshape(N, 1), W1, b1.reshape(1, OUT), W2, b2.reshape(1, OUT))

    eat = edge_attr.T
    w1e = W1[2 * D:2 * D + DE]

    EBLK = 16000
    ew2 = pl.pallas_call(
        _ew_body,
        grid=(E // EBLK,),
        in_specs=[
            pl.BlockSpec((DE, EBLK), lambda i: (0, i)),
            pl.BlockSpec((DE, OUT), lambda i: (0, 0)),
        ],
        out_specs=pl.BlockSpec((EBLK, OUT), lambda i: (i, 0)),
        out_shape=jax.ShapeDtypeStruct((E, OUT), jnp.float32),
    )(eat, w1e)

    mesh = plsc.VectorSubcoreMesh(core_axis_name="c", subcore_axis_name="s")
    g = pl.kernel(
        _sc_body,
        out_type=jax.ShapeDtypeStruct((NC, N, H), jnp.float32),
        mesh=mesh,
        compiler_params=pltpu.CompilerParams(use_tc_tiling_on_sc=False),
        scratch_types=[
            pltpu.VMEM((NCHUNK, CHUNK), jnp.int32),
            pltpu.VMEM((NCHUNK, CHUNK), jnp.int32),
            [pltpu.VMEM((CHUNK, H), jnp.float32)] * 2,
            [pltpu.VMEM((CHUNK, H), jnp.float32)] * 2,
            [pltpu.VMEM((CHUNK, H), jnp.float32)] * 2,
            [pltpu.VMEM((CHUNK, H), jnp.float32)] * 2,
            pltpu.VMEM((RD, H), jnp.float32),
            pltpu.VMEM_SHARED((N, H), jnp.float32),
            [pltpu.SemaphoreType.DMA] * 2,
            [pltpu.SemaphoreType.DMA] * 2,
            [pltpu.SemaphoreType.DMA] * 2,
            [pltpu.SemaphoreType.DMA] * 2,
        ],
    )(p, q, ew2, dst, src)

    out = pl.pallas_call(
        _final_body,
        out_shape=jax.ShapeDtypeStruct((N, OUT), jnp.float32),
    )(g, r, W2[D:D + OUT])
    return out


# speedup vs baseline: 1.6335x; 1.1598x over previous
import functools

import jax
import jax.numpy as jnp
from jax import lax
from jax.experimental import pallas as pl
from jax.experimental.pallas import tpu as pltpu
from jax.experimental.pallas import tpu_sc as plsc

N = 10000
E = 320000
D = 128
DE = 16
DU = 16
G = 128
OUT = 128
H = OUT // 2

NC = 2
NS = 16
EPT = E // NS
CHUNK = 40
NCHUNK = EPT // CHUNK
NBUF = 3
RD = 40
NRD = N // RD

_HIGH = jax.lax.Precision.HIGHEST


def _precompute_body(x_ref, u_ref, batch_ref, wx_ref, wu_ref, b1_ref, b2_ref,
                     p_ref, q_ref, r_ref):
    x = x_ref[...]
    bat = batch_ref[...]
    gidx = lax.broadcasted_iota(jnp.int32, (x.shape[0], G), 1)
    onehot = (bat == gidx).astype(jnp.float32)
    uu = jnp.dot(onehot, u_ref[...], preferred_element_type=jnp.float32)
    xc = jnp.dot(x, wx_ref[...], preferred_element_type=jnp.float32)
    uc = jnp.dot(uu, wu_ref[...], preferred_element_type=jnp.float32)
    p = xc[:, :D] + uc[:, :D] + b1_ref[...]
    q = xc[:, D:2 * D]
    r = xc[:, 2 * D:] + uc[:, D:] + b2_ref[...]
    p_ref[0] = p[:, :H]
    p_ref[1] = p[:, H:]
    q_ref[0] = q[:, :H]
    q_ref[1] = q[:, H:]
    r_ref[...] = r


def _ew_body(eat8_ref, w1e_ref, ew_ref):
    dn = (((0,), (0,)), ((), ()))
    ew_ref[...] = lax.dot_general(eat8_ref[...], w1e_ref[...], dn,
                                  preferred_element_type=jnp.float32)


def _final_body(g_ref, r_ref, w2b_ref, out_ref):
    grouped = jnp.concatenate([g_ref[0], g_ref[1]], axis=1)
    acc = jnp.dot(grouped, w2b_ref[...], preferred_element_type=jnp.float32,
                  precision=_HIGH)
    out_ref[...] = jnp.maximum(acc + r_ref[...], 0.0)


def _sc_body(p_hbm, q_hbm, ew_hbm, dst_hbm, src_hbm, g_hbm,
             idxd, idxs, buf_e, buf_p, buf_q, buf_o, zbuf, acc,
             sem_e, sem_p, sem_q, sem_s):
    c = lax.axis_index("c")
    s = lax.axis_index("s")

    def _zrow(r, carry):
        for cc in range(H // 16):
            zbuf[r, pl.ds(cc * 16, 16)] = jnp.zeros((16,), jnp.float32)
        return carry
    lax.fori_loop(0, RD, _zrow, 0)
    for t in range((NRD + NS - 1) // NS):
        cidx = t * NS + s
        @pl.when(cidx < NRD)
        def _():
            pltpu.sync_copy(zbuf, acc.at[pl.ds(cidx * RD, RD)])
    plsc.subcore_barrier()

    pltpu.sync_copy(dst_hbm.at[s], idxd)
    pltpu.sync_copy(src_hbm.at[s], idxs)

    pc = p_hbm.at[c]
    qc = q_hbm.at[c]

    def _issue_in(j, b):
        ebase = s * EPT + j * CHUNK
        pltpu.async_copy(
            ew_hbm.at[pl.ds(ebase, CHUNK), pl.ds(c * H, H)], buf_e[b], sem_e[b])
        pltpu.async_copy(pc.at[idxd.at[j]], buf_p[b], sem_p[b])
        pltpu.async_copy(qc.at[idxs.at[j]], buf_q[b], sem_q[b])

    def _wait_in(b):
        pltpu.make_async_copy(pc.at[pl.ds(0, CHUNK)], buf_e[b], sem_e[b]).wait()
        pltpu.make_async_copy(pc.at[pl.ds(0, CHUNK)], buf_p[b], sem_p[b]).wait()
        pltpu.make_async_copy(pc.at[pl.ds(0, CHUNK)], buf_q[b], sem_q[b]).wait()

    for b in range(NBUF):
        _issue_in(b, b)

    def _outer(j0, carry):
        for b in range(NBUF):
            j = j0 * NBUF + b

            @pl.when(j < NCHUNK)
            def _():
                _wait_in(b)

                @pl.when(j0 > 0)
                def _():
                    pltpu.make_async_copy(pc.at[pl.ds(0, CHUNK)], buf_o[b],
                                          sem_s[b]).wait()

                def _row(e, carry2):
                    for cc in range(H // 16):
                        sl = pl.ds(cc * 16, 16)
                        v = (buf_e[b][e, sl]
                             + buf_p[b][e, sl] + buf_q[b][e, sl])
                        buf_o[b][e, sl] = jnp.maximum(v, 0.0)
                    return carry2
                lax.fori_loop(0, CHUNK, _row, 0)

                pltpu.async_copy(buf_o[b], acc.at[idxd.at[j]], sem_s[b],
                                 add=True)

                @pl.when(j + NBUF < NCHUNK)
                def _():
                    _issue_in(j + NBUF, b)
        return carry
    lax.fori_loop(0, (NCHUNK + NBUF - 1) // NBUF, _outer, 0)
    for b in range(NBUF):
        pltpu.make_async_copy(pc.at[pl.ds(0, CHUNK)], buf_o[b], sem_s[b]).wait()
    plsc.subcore_barrier()

    for t in range((NRD + NS - 1) // NS):
        cidx = t * NS + s
        @pl.when(cidx < NRD)
        def _():
            pltpu.sync_copy(acc.at[pl.ds(cidx * RD, RD)], zbuf)
            pltpu.sync_copy(zbuf, g_hbm.at[c, pl.ds(cidx * RD, RD)])


@jax.jit
def kernel(x, edge_index, edge_attr, u, batch, W1, b1, W2, b2):
    src = edge_index[0].reshape(NS, NCHUNK, CHUNK)
    dst = edge_index[1].reshape(NS, NCHUNK, CHUNK)

    wx = jnp.concatenate([W1[:D], W1[D:2 * D], W2[:D]], axis=1)
    wu = jnp.concatenate([W1[2 * D + DE:], W2[D + OUT:]], axis=1)

    NBLK = 1000
    p, q, r = pl.pallas_call(
        _precompute_body,
        grid=(N // NBLK,),
        in_specs=[
            pl.BlockSpec((NBLK, D), lambda i: (i, 0)),
            pl.BlockSpec((G, DU), lambda i: (0, 0)),
            pl.BlockSpec((NBLK, 1), lambda i: (i, 0)),
            pl.BlockSpec((D, 3 * OUT), lambda i: (0, 0)),
            pl.BlockSpec((DU, 2 * OUT), lambda i: (0, 0)),
            pl.BlockSpec((1, OUT), lambda i: (0, 0)),
            pl.BlockSpec((1, OUT), lambda i: (0, 0)),
        ],
        out_specs=[
            pl.BlockSpec((NC, NBLK, H), lambda i: (0, i, 0)),
            pl.BlockSpec((NC, NBLK, H), lambda i: (0, i, 0)),
            pl.BlockSpec((NBLK, OUT), lambda i: (i, 0)),
        ],
        out_shape=[
            jax.ShapeDtypeStruct((NC, N, H), jnp.float32),
            jax.ShapeDtypeStruct((NC, N, H), jnp.float32),
            jax.ShapeDtypeStruct((N, OUT), jnp.float32),
        ],
    )(x, u, batch.reshape(N, 1), wx, wu, b1.reshape(1, OUT),
      b2.reshape(1, OUT))

    eat = edge_attr.T
    w1e = W1[2 * D:2 * D + DE]

    EBLK = 16000
    ew2 = pl.pallas_call(
        _ew_body,
        grid=(E // EBLK,),
        in_specs=[
            pl.BlockSpec((DE, EBLK), lambda i: (0, i)),
            pl.BlockSpec((DE, OUT), lambda i: (0, 0)),
        ],
        out_specs=pl.BlockSpec((EBLK, OUT), lambda i: (i, 0)),
        out_shape=jax.ShapeDtypeStruct((E, OUT), jnp.float32),
    )(eat, w1e)

    mesh = plsc.VectorSubcoreMesh(core_axis_name="c", subcore_axis_name="s")
    g = pl.kernel(
        _sc_body,
        out_type=jax.ShapeDtypeStruct((NC, N, H), jnp.float32),
        mesh=mesh,
        compiler_params=pltpu.CompilerParams(use_tc_tiling_on_sc=False),
        scratch_types=[
            pltpu.VMEM((NCHUNK, CHUNK), jnp.int32),
            pltpu.VMEM((NCHUNK, CHUNK), jnp.int32),
            [pltpu.VMEM((CHUNK, H), jnp.float32)] * NBUF,
            [pltpu.VMEM((CHUNK, H), jnp.float32)] * NBUF,
            [pltpu.VMEM((CHUNK, H), jnp.float32)] * NBUF,
            [pltpu.VMEM((CHUNK, H), jnp.float32)] * NBUF,
            pltpu.VMEM((RD, H), jnp.float32),
            pltpu.VMEM_SHARED((N, H), jnp.float32),
            [pltpu.SemaphoreType.DMA] * NBUF,
            [pltpu.SemaphoreType.DMA] * NBUF,
            [pltpu.SemaphoreType.DMA] * NBUF,
            [pltpu.SemaphoreType.DMA] * NBUF,
        ],
    )(p, q, ew2, dst, src)

    out = pl.pallas_call(
        _final_body,
        out_shape=jax.ShapeDtypeStruct((N, OUT), jnp.float32),
    )(g, r, W2[D:D + OUT])
    return out
